# baseline (speedup 1.0000x reference)
import jax
import jax.numpy as jnp
from jax.experimental import pallas as pl
from jax.experimental.pallas import tpu as pltpu

_ORDER = tuple(range(17))

_B, _N, _J, _D = 32, 243, 17, 256
_TOTAL = _B * _N * _J * _D
_NCH = 8
_CHUNK = _TOTAL // _NCH


def _dma_copy(x_hbm, o_hbm, sem):
    for g in range(_NCH):
        pltpu.make_async_copy(
            x_hbm.at[pl.ds(g * _CHUNK, _CHUNK)],
            o_hbm.at[pl.ds(g * _CHUNK, _CHUNK)],
            sem,
        ).start()
    for g in range(_NCH):
        pltpu.make_async_copy(
            x_hbm.at[pl.ds(g * _CHUNK, _CHUNK)],
            o_hbm.at[pl.ds(g * _CHUNK, _CHUNK)],
            sem,
        ).wait()


def kernel(x):
    flat = x.reshape(_TOTAL)
    out = pl.pallas_call(
        _dma_copy,
        in_specs=[pl.BlockSpec(memory_space=pl.ANY)],
        out_specs=pl.BlockSpec(memory_space=pl.ANY),
        out_shape=jax.ShapeDtypeStruct((_TOTAL,), jnp.float32),
        scratch_shapes=[pltpu.SemaphoreType.DMA],
    )(flat)
    return out.reshape(_B, _N, _J, _D)


# speedup vs baseline: 16.8246x; 16.8246x over previous
import jax
import jax.numpy as jnp
from jax.experimental import pallas as pl

_ORDER = (0, 1, 2, 3, 4, 5, 6, 7, 8, 9, 10, 11, 12, 13, 14, 15, 16)
_IS_IDENTITY = _ORDER == tuple(range(len(_ORDER)))


def _reorder_block(x_ref, o_ref):
    if _IS_IDENTITY:
        o_ref[...] = x_ref[...]
    else:
        for j, s in enumerate(_ORDER):
            o_ref[:, :, j, :] = x_ref[:, :, s, :]


def kernel(x):
    b, n, j, d = x.shape
    grid = (b, 3)
    bn = n // 3
    return pl.pallas_call(
        _reorder_block,
        grid=grid,
        in_specs=[pl.BlockSpec((1, bn, j, d), lambda i, t: (i, t, 0, 0))],
        out_specs=pl.BlockSpec((1, bn, j, d), lambda i, t: (i, t, 0, 0)),
        out_shape=jax.ShapeDtypeStruct((b, n, j, d), x.dtype),
    )(x)
